# Initial kernel scaffold; baseline (speedup 1.0000x reference)
import jax
import jax.numpy as jnp
from jax.experimental import pallas as pl


def kernel(x, edge_index, W1, b1, W2, b2, W3, b3, Wo, bo):
    raise NotImplementedError("write your pallas kernel here")


# speedup vs baseline: 11.2119x; 11.2119x over previous
import functools

import jax
import jax.numpy as jnp
from jax import lax
from jax.experimental import pallas as pl
from jax.experimental.pallas import tpu as pltpu
from jax.experimental.pallas import tpu_sc as plsc

N = 10000
E = 320000
D = 128
D_OUT = 64

NC = 2
NS = 16
CH = 80
EPW = E // (NC * NS)
NCHUNK = EPW // CH

RPT = 624
NZCH = 8


def _chunk_starts(s):
    return [jnp.minimum(s * RPT + t * CH, N - CH) for t in range(NZCH)]


_mesh = plsc.VectorSubcoreMesh(core_axis_name="c", subcore_axis_name="s")


@functools.partial(
    pl.kernel,
    out_type=jax.ShapeDtypeStruct((NC * N,), jnp.float32),
    mesh=_mesh,
    scratch_types=[
        pltpu.VMEM((CH,), jnp.int32),
        pltpu.VMEM((CH,), jnp.float32),
        pltpu.VMEM((CH,), jnp.float32),
        pltpu.VMEM_SHARED((N,), jnp.float32),
        pltpu.SemaphoreType.DMA,
    ],
)
def _deg_kernel(dst_hbm, out_hbm, dst_v, ones_v, zero_v, acc, sem):
    c = lax.axis_index("c")
    s = lax.axis_index("s")

    for j in range(CH // 16):
        ones_v[pl.ds(j * 16, 16)] = jnp.ones((16,), jnp.float32)
        zero_v[pl.ds(j * 16, 16)] = jnp.zeros((16,), jnp.float32)

    for st in _chunk_starts(s):
        pltpu.sync_copy(zero_v, acc.at[pl.ds(st, CH)])

    plsc.subcore_barrier()

    base = (c * NS + s) * EPW

    def body(k, carry):
        pltpu.sync_copy(dst_hbm.at[pl.ds(base + k * CH, CH)], dst_v)
        pltpu.sync_copy(ones_v, acc.at[dst_v], add=True)
        return carry

    lax.fori_loop(0, NCHUNK, body, 0)
    plsc.subcore_barrier()

    for st in _chunk_starts(s):
        pltpu.sync_copy(acc.at[pl.ds(st, CH)], zero_v)
        pltpu.sync_copy(zero_v, out_hbm.at[pl.ds(c * N + st, CH)])


@functools.partial(
    pl.kernel,
    out_type=jax.ShapeDtypeStruct((NC, N, D), jnp.float32),
    mesh=_mesh,
    scratch_types=[
        pltpu.VMEM((CH,), jnp.int32),
        pltpu.VMEM((CH,), jnp.int32),
        pltpu.VMEM((CH, D), jnp.float32),
        pltpu.VMEM_SHARED((N, D), jnp.float32),
        pltpu.SemaphoreType.DMA,
    ],
)
def _prop_kernel(h_hbm, src_hbm, dst_hbm, out_hbm,
                 src_v, dst_v, rows_v, acc, sem):
    c = lax.axis_index("c")
    s = lax.axis_index("s")

    def zrow(k, carry):
        for j in range(D // 16):
            rows_v[k, pl.ds(j * 16, 16)] = jnp.zeros((16,), jnp.float32)
        return carry

    lax.fori_loop(0, CH, zrow, 0)

    for st in _chunk_starts(s):
        pltpu.sync_copy(rows_v, acc.at[pl.ds(st, CH)])

    plsc.subcore_barrier()

    base = (c * NS + s) * EPW

    def body(k, carry):
        off = base + k * CH
        pltpu.sync_copy(src_hbm.at[pl.ds(off, CH)], src_v)
        pltpu.sync_copy(dst_hbm.at[pl.ds(off, CH)], dst_v)
        pltpu.async_copy(h_hbm.at[src_v], rows_v, sem).wait()
        pltpu.sync_copy(rows_v, acc.at[dst_v], add=True)
        return carry

    lax.fori_loop(0, NCHUNK, body, 0)
    plsc.subcore_barrier()

    for st in _chunk_starts(s):
        pltpu.sync_copy(acc.at[pl.ds(st, CH)], rows_v)
        pltpu.sync_copy(rows_v, out_hbm.at[c, pl.ds(st, CH)])


_BR = 1000
_GRID = N // _BR


def _mm_scale_body(x_ref, w_ref, d0_ref, d1_ref, h_ref, inv_ref):
    inv = lax.rsqrt(d0_ref[...] + d1_ref[...] + 1.0)
    h = jnp.dot(x_ref[...], w_ref[...], preferred_element_type=jnp.float32)
    h_ref[...] = h * inv
    inv_ref[...] = inv


def _first_layer(x, W1, d0, d1):
    return pl.pallas_call(
        _mm_scale_body,
        grid=(_GRID,),
        in_specs=[
            pl.BlockSpec((_BR, D), lambda i: (i, 0)),
            pl.BlockSpec((D, D), lambda i: (0, 0)),
            pl.BlockSpec((_BR, 1), lambda i: (i, 0)),
            pl.BlockSpec((_BR, 1), lambda i: (i, 0)),
        ],
        out_specs=[
            pl.BlockSpec((_BR, D), lambda i: (i, 0)),
            pl.BlockSpec((_BR, 1), lambda i: (i, 0)),
        ],
        out_shape=[
            jax.ShapeDtypeStruct((N, D), jnp.float32),
            jax.ShapeDtypeStruct((N, 1), jnp.float32),
        ],
    )(x, W1, d0, d1)


def _combine_body(final, s_parts_ref0, s_parts_ref1, h_ref, inv_ref, b_ref,
                  w_ref, bo_ref, o_ref):
    s_tot = s_parts_ref0[0] + s_parts_ref1[0] + h_ref[...]
    z = jnp.maximum(s_tot * inv_ref[...] + b_ref[...], 0.0)
    o = jnp.dot(z, w_ref[...], preferred_element_type=jnp.float32)
    if final:
        o_ref[...] = o + bo_ref[...]
    else:
        o_ref[...] = o * inv_ref[...]


def _combine(s_parts, h, inv, b, w, bo, final):
    d_out = w.shape[1]
    return pl.pallas_call(
        functools.partial(_combine_body, final),
        grid=(_GRID,),
        in_specs=[
            pl.BlockSpec((1, _BR, D), lambda i: (0, i, 0)),
            pl.BlockSpec((1, _BR, D), lambda i: (1, i, 0)),
            pl.BlockSpec((_BR, D), lambda i: (i, 0)),
            pl.BlockSpec((_BR, 1), lambda i: (i, 0)),
            pl.BlockSpec((1, D), lambda i: (0, 0)),
            pl.BlockSpec((D, d_out), lambda i: (0, 0)),
            pl.BlockSpec((1, d_out), lambda i: (0, 0)),
        ],
        out_specs=pl.BlockSpec((_BR, d_out), lambda i: (i, 0)),
        out_shape=jax.ShapeDtypeStruct((N, d_out), jnp.float32),
    )(s_parts, s_parts, h, inv, b, w, bo)


def kernel(x, edge_index, W1, b1, W2, b2, W3, b3, Wo, bo):
    src = edge_index[0].astype(jnp.int32)
    dst = edge_index[1].astype(jnp.int32)

    deg_parts = _deg_kernel(dst)
    d0 = deg_parts[:N][:, None]
    d1 = deg_parts[N:][:, None]

    h1, inv = _first_layer(x, W1, d0, d1)

    b1r = b1[None, :]
    b2r = b2[None, :]
    b3r = b3[None, :]
    bor = bo[None, :]
    dummy = jnp.zeros((1, D), jnp.float32)

    s1 = _prop_kernel(h1, src, dst)
    h2 = _combine(s1, h1, inv, b1r, W2, dummy, final=False)
    s2 = _prop_kernel(h2, src, dst)
    h3 = _combine(s2, h2, inv, b2r, W3, dummy, final=False)
    s3 = _prop_kernel(h3, src, dst)
    out = _combine(s3, h3, inv, b3r, Wo, bor, final=True)
    return out


# speedup vs baseline: 19.4032x; 1.7306x over previous
import functools

import jax
import jax.numpy as jnp
from jax import lax
from jax.experimental import pallas as pl
from jax.experimental.pallas import tpu as pltpu
from jax.experimental.pallas import tpu_sc as plsc

N = 10000
E = 320000
D = 128
D_OUT = 64

NC = 2
NS = 16
CH = 80
EPW = E // (NC * NS)
NCHUNK = EPW // CH

RPT = 624
NZCH = 8


def _chunk_starts(s):
    return [jnp.minimum(s * RPT + t * CH, N - CH) for t in range(NZCH)]


_mesh = plsc.VectorSubcoreMesh(core_axis_name="c", subcore_axis_name="s")


@functools.partial(
    pl.kernel,
    out_type=jax.ShapeDtypeStruct((NC * N,), jnp.float32),
    mesh=_mesh,
    scratch_types=[
        pltpu.VMEM((CH,), jnp.int32),
        pltpu.VMEM((CH,), jnp.float32),
        pltpu.VMEM((CH,), jnp.float32),
        pltpu.VMEM_SHARED((N,), jnp.float32),
        pltpu.SemaphoreType.DMA,
    ],
)
def _deg_kernel(dst_hbm, out_hbm, dst_v, ones_v, zero_v, acc, sem):
    c = lax.axis_index("c")
    s = lax.axis_index("s")

    for j in range(CH // 16):
        ones_v[pl.ds(j * 16, 16)] = jnp.ones((16,), jnp.float32)
        zero_v[pl.ds(j * 16, 16)] = jnp.zeros((16,), jnp.float32)

    for st in _chunk_starts(s):
        pltpu.sync_copy(zero_v, acc.at[pl.ds(st, CH)])

    plsc.subcore_barrier()

    base = (c * NS + s) * EPW

    def body(k, carry):
        pltpu.sync_copy(dst_hbm.at[pl.ds(base + k * CH, CH)], dst_v)
        pltpu.sync_copy(ones_v, acc.at[dst_v], add=True)
        return carry

    lax.fori_loop(0, NCHUNK, body, 0)
    plsc.subcore_barrier()

    for st in _chunk_starts(s):
        pltpu.sync_copy(acc.at[pl.ds(st, CH)], zero_v)
        pltpu.sync_copy(zero_v, out_hbm.at[pl.ds(c * N + st, CH)])


@functools.partial(
    pl.kernel,
    out_type=jax.ShapeDtypeStruct((NC, N, D), jnp.float32),
    mesh=_mesh,
    scratch_types=[
        pltpu.VMEM((CH,), jnp.int32),
        pltpu.VMEM((CH,), jnp.int32),
        pltpu.VMEM((CH,), jnp.int32),
        pltpu.VMEM((CH,), jnp.int32),
        pltpu.VMEM((CH, D), jnp.float32),
        pltpu.VMEM((CH, D), jnp.float32),
        pltpu.VMEM_SHARED((N, D), jnp.float32),
        pltpu.SemaphoreType.DMA,
        pltpu.SemaphoreType.DMA,
    ],
)
def _prop_kernel(h_hbm, src_hbm, dst_hbm, out_hbm,
                 src_a, dst_a, src_b, dst_b, rows_a, rows_b, acc,
                 sem_g, sem_i):
    c = lax.axis_index("c")
    s = lax.axis_index("s")

    def zrow(k, carry):
        for j in range(D // 16):
            rows_a[k, pl.ds(j * 16, 16)] = jnp.zeros((16,), jnp.float32)
        return carry

    lax.fori_loop(0, CH, zrow, 0)

    for st in _chunk_starts(s):
        pltpu.sync_copy(rows_a, acc.at[pl.ds(st, CH)])

    plsc.subcore_barrier()

    base = (c * NS + s) * EPW

    def idx_start(k, sv, dv):
        off = base + k * CH
        pltpu.async_copy(src_hbm.at[pl.ds(off, CH)], sv, sem_i)
        pltpu.async_copy(dst_hbm.at[pl.ds(off, CH)], dv, sem_i)

    def idx_wait(sv, dv):
        pltpu.make_async_copy(src_hbm.at[pl.ds(0, CH)], sv, sem_i).wait()
        pltpu.make_async_copy(dst_hbm.at[pl.ds(0, CH)], dv, sem_i).wait()

    def gather_start(sv, rv):
        pltpu.async_copy(h_hbm.at[sv], rv, sem_g)

    def gather_wait(sv, rv):
        pltpu.make_async_copy(h_hbm.at[sv], rv, sem_g).wait()

    pltpu.sync_copy(src_hbm.at[pl.ds(base, CH)], src_a)
    pltpu.sync_copy(dst_hbm.at[pl.ds(base, CH)], dst_a)
    gather_start(src_a, rows_a)
    idx_start(1, src_b, dst_b)

    def pair(g, carry):
        k0 = 2 * g
        gather_wait(src_a, rows_a)
        idx_wait(src_b, dst_b)
        gather_start(src_b, rows_b)
        pltpu.sync_copy(rows_a, acc.at[dst_a], add=True)
        idx_start(k0 + 2, src_a, dst_a)
        gather_wait(src_b, rows_b)
        idx_wait(src_a, dst_a)
        gather_start(src_a, rows_a)
        pltpu.sync_copy(rows_b, acc.at[dst_b], add=True)

        @pl.when(g < (NCHUNK - 1) // 2 - 1)
        def _():
            idx_start(k0 + 3, src_b, dst_b)

        return carry

    lax.fori_loop(0, (NCHUNK - 1) // 2, pair, 0)

    gather_wait(src_a, rows_a)
    pltpu.sync_copy(rows_a, acc.at[dst_a], add=True)

    plsc.subcore_barrier()

    for st in _chunk_starts(s):
        pltpu.sync_copy(acc.at[pl.ds(st, CH)], rows_a)
        pltpu.sync_copy(rows_a, out_hbm.at[c, pl.ds(st, CH)])


_BR = 1000
_GRID = N // _BR


def _mm_scale_body(x_ref, w_ref, d0_ref, d1_ref, h_ref, inv_ref):
    inv = lax.rsqrt(d0_ref[...] + d1_ref[...] + 1.0)
    h = jnp.dot(x_ref[...], w_ref[...], preferred_element_type=jnp.float32)
    h_ref[...] = h * inv
    inv_ref[...] = inv


def _first_layer(x, W1, d0, d1):
    return pl.pallas_call(
        _mm_scale_body,
        grid=(_GRID,),
        in_specs=[
            pl.BlockSpec((_BR, D), lambda i: (i, 0)),
            pl.BlockSpec((D, D), lambda i: (0, 0)),
            pl.BlockSpec((_BR, 1), lambda i: (i, 0)),
            pl.BlockSpec((_BR, 1), lambda i: (i, 0)),
        ],
        out_specs=[
            pl.BlockSpec((_BR, D), lambda i: (i, 0)),
            pl.BlockSpec((_BR, 1), lambda i: (i, 0)),
        ],
        out_shape=[
            jax.ShapeDtypeStruct((N, D), jnp.float32),
            jax.ShapeDtypeStruct((N, 1), jnp.float32),
        ],
    )(x, W1, d0, d1)


def _combine_body(final, s_parts_ref0, s_parts_ref1, h_ref, inv_ref, b_ref,
                  w_ref, bo_ref, o_ref):
    s_tot = s_parts_ref0[0] + s_parts_ref1[0] + h_ref[...]
    z = jnp.maximum(s_tot * inv_ref[...] + b_ref[...], 0.0)
    o = jnp.dot(z, w_ref[...], preferred_element_type=jnp.float32)
    if final:
        o_ref[...] = o + bo_ref[...]
    else:
        o_ref[...] = o * inv_ref[...]


def _combine(s_parts, h, inv, b, w, bo, final):
    d_out = w.shape[1]
    return pl.pallas_call(
        functools.partial(_combine_body, final),
        grid=(_GRID,),
        in_specs=[
            pl.BlockSpec((1, _BR, D), lambda i: (0, i, 0)),
            pl.BlockSpec((1, _BR, D), lambda i: (1, i, 0)),
            pl.BlockSpec((_BR, D), lambda i: (i, 0)),
            pl.BlockSpec((_BR, 1), lambda i: (i, 0)),
            pl.BlockSpec((1, D), lambda i: (0, 0)),
            pl.BlockSpec((D, d_out), lambda i: (0, 0)),
            pl.BlockSpec((1, d_out), lambda i: (0, 0)),
        ],
        out_specs=pl.BlockSpec((_BR, d_out), lambda i: (i, 0)),
        out_shape=jax.ShapeDtypeStruct((N, d_out), jnp.float32),
    )(s_parts, s_parts, h, inv, b, w, bo)


def kernel(x, edge_index, W1, b1, W2, b2, W3, b3, Wo, bo):
    src = edge_index[0].astype(jnp.int32)
    dst = edge_index[1].astype(jnp.int32)

    deg_parts = _deg_kernel(dst)
    d0 = deg_parts[:N][:, None]
    d1 = deg_parts[N:][:, None]

    h1, inv = _first_layer(x, W1, d0, d1)

    b1r = b1[None, :]
    b2r = b2[None, :]
    b3r = b3[None, :]
    bor = bo[None, :]
    dummy = jnp.zeros((1, D), jnp.float32)

    s1 = _prop_kernel(h1, src, dst)
    h2 = _combine(s1, h1, inv, b1r, W2, dummy, final=False)
    s2 = _prop_kernel(h2, src, dst)
    h3 = _combine(s2, h2, inv, b2r, W3, dummy, final=False)
    s3 = _prop_kernel(h3, src, dst)
    out = _combine(s3, h3, inv, b3r, Wo, bor, final=True)
    return out


# speedup vs baseline: 20.2826x; 1.0453x over previous
import functools

import jax
import jax.numpy as jnp
from jax import lax
from jax.experimental import pallas as pl
from jax.experimental.pallas import tpu as pltpu
from jax.experimental.pallas import tpu_sc as plsc

N = 10000
E = 320000
D = 128
D_OUT = 64

NC = 2
NS = 16
CH = 80
EPW = E // (NC * NS)
NCHUNK = EPW // CH

RPT = 624
NZCH = 8


def _chunk_starts(s):
    return [jnp.minimum(s * RPT + t * CH, N - CH) for t in range(NZCH)]


_mesh = plsc.VectorSubcoreMesh(core_axis_name="c", subcore_axis_name="s")


@functools.partial(
    pl.kernel,
    out_type=jax.ShapeDtypeStruct((NC * N,), jnp.float32),
    mesh=_mesh,
    scratch_types=[
        pltpu.VMEM((CH,), jnp.int32),
        pltpu.VMEM((CH,), jnp.int32),
        pltpu.VMEM((CH,), jnp.float32),
        pltpu.VMEM((CH,), jnp.float32),
        pltpu.VMEM_SHARED((N,), jnp.float32),
        pltpu.SemaphoreType.DMA,
    ],
)
def _deg_kernel(dst_hbm, out_hbm, dst_a, dst_b, ones_v, zero_v, acc, sem):
    c = lax.axis_index("c")
    s = lax.axis_index("s")

    for j in range(CH // 16):
        ones_v[pl.ds(j * 16, 16)] = jnp.ones((16,), jnp.float32)
        zero_v[pl.ds(j * 16, 16)] = jnp.zeros((16,), jnp.float32)

    for st in _chunk_starts(s):
        pltpu.sync_copy(zero_v, acc.at[pl.ds(st, CH)])

    plsc.subcore_barrier()

    base = (c * NS + s) * EPW

    def idx_start(k, dv):
        pltpu.async_copy(dst_hbm.at[pl.ds(base + k * CH, CH)], dv, sem)

    def idx_wait(dv):
        pltpu.make_async_copy(dst_hbm.at[pl.ds(0, CH)], dv, sem).wait()

    idx_start(0, dst_a)
    idx_start(1, dst_b)

    def pair(g, carry):
        k0 = 2 * g
        idx_wait(dst_a)
        pltpu.sync_copy(ones_v, acc.at[dst_a], add=True)
        idx_start(k0 + 2, dst_a)
        idx_wait(dst_b)
        pltpu.sync_copy(ones_v, acc.at[dst_b], add=True)

        @pl.when(g < (NCHUNK - 1) // 2 - 1)
        def _():
            idx_start(k0 + 3, dst_b)

        return carry

    lax.fori_loop(0, (NCHUNK - 1) // 2, pair, 0)

    idx_wait(dst_a)
    pltpu.sync_copy(ones_v, acc.at[dst_a], add=True)
    plsc.subcore_barrier()

    for st in _chunk_starts(s):
        pltpu.sync_copy(acc.at[pl.ds(st, CH)], zero_v)
        pltpu.sync_copy(zero_v, out_hbm.at[pl.ds(c * N + st, CH)])


@functools.partial(
    pl.kernel,
    out_type=jax.ShapeDtypeStruct((NC, N, D), jnp.float32),
    mesh=_mesh,
    scratch_types=[
        pltpu.VMEM((CH,), jnp.int32),
        pltpu.VMEM((CH,), jnp.int32),
        pltpu.VMEM((CH,), jnp.int32),
        pltpu.VMEM((CH,), jnp.int32),
        pltpu.VMEM((CH, D), jnp.float32),
        pltpu.VMEM((CH, D), jnp.float32),
        pltpu.VMEM_SHARED((N, D), jnp.float32),
        pltpu.SemaphoreType.DMA,
        pltpu.SemaphoreType.DMA,
    ],
)
def _prop_kernel(h_hbm, src_hbm, dst_hbm, out_hbm,
                 src_a, dst_a, src_b, dst_b, rows_a, rows_b, acc,
                 sem_g, sem_i):
    c = lax.axis_index("c")
    s = lax.axis_index("s")

    def zrow(k, carry):
        for j in range(D // 16):
            rows_a[k, pl.ds(j * 16, 16)] = jnp.zeros((16,), jnp.float32)
        return carry

    lax.fori_loop(0, CH, zrow, 0)

    for st in _chunk_starts(s):
        pltpu.sync_copy(rows_a, acc.at[pl.ds(st, CH)])

    plsc.subcore_barrier()

    base = (c * NS + s) * EPW

    def idx_start(k, sv, dv):
        off = base + k * CH
        pltpu.async_copy(src_hbm.at[pl.ds(off, CH)], sv, sem_i)
        pltpu.async_copy(dst_hbm.at[pl.ds(off, CH)], dv, sem_i)

    def idx_wait(sv, dv):
        pltpu.make_async_copy(src_hbm.at[pl.ds(0, CH)], sv, sem_i).wait()
        pltpu.make_async_copy(dst_hbm.at[pl.ds(0, CH)], dv, sem_i).wait()

    def gather_start(sv, rv):
        pltpu.async_copy(h_hbm.at[sv], rv, sem_g)

    def gather_wait(sv, rv):
        pltpu.make_async_copy(h_hbm.at[sv], rv, sem_g).wait()

    pltpu.sync_copy(src_hbm.at[pl.ds(base, CH)], src_a)
    pltpu.sync_copy(dst_hbm.at[pl.ds(base, CH)], dst_a)
    gather_start(src_a, rows_a)
    idx_start(1, src_b, dst_b)

    def pair(g, carry):
        k0 = 2 * g
        gather_wait(src_a, rows_a)
        idx_wait(src_b, dst_b)
        gather_start(src_b, rows_b)
        pltpu.sync_copy(rows_a, acc.at[dst_a], add=True)
        idx_start(k0 + 2, src_a, dst_a)
        gather_wait(src_b, rows_b)
        idx_wait(src_a, dst_a)
        gather_start(src_a, rows_a)
        pltpu.sync_copy(rows_b, acc.at[dst_b], add=True)

        @pl.when(g < (NCHUNK - 1) // 2 - 1)
        def _():
            idx_start(k0 + 3, src_b, dst_b)

        return carry

    lax.fori_loop(0, (NCHUNK - 1) // 2, pair, 0)

    gather_wait(src_a, rows_a)
    pltpu.sync_copy(rows_a, acc.at[dst_a], add=True)

    plsc.subcore_barrier()

    for st in _chunk_starts(s):
        pltpu.sync_copy(acc.at[pl.ds(st, CH)], rows_a)
        pltpu.sync_copy(rows_a, out_hbm.at[c, pl.ds(st, CH)])


_BR = 1000
_GRID = N // _BR


def _mm_scale_body(x_ref, w_ref, d0_ref, d1_ref, h_ref, inv_ref):
    inv = lax.rsqrt(d0_ref[...] + d1_ref[...] + 1.0)
    h = jnp.dot(x_ref[...], w_ref[...], preferred_element_type=jnp.float32)
    h_ref[...] = h * inv
    inv_ref[...] = inv


def _first_layer(x, W1, d0, d1):
    return pl.pallas_call(
        _mm_scale_body,
        grid=(_GRID,),
        in_specs=[
            pl.BlockSpec((_BR, D), lambda i: (i, 0)),
            pl.BlockSpec((D, D), lambda i: (0, 0)),
            pl.BlockSpec((_BR, 1), lambda i: (i, 0)),
            pl.BlockSpec((_BR, 1), lambda i: (i, 0)),
        ],
        out_specs=[
            pl.BlockSpec((_BR, D), lambda i: (i, 0)),
            pl.BlockSpec((_BR, 1), lambda i: (i, 0)),
        ],
        out_shape=[
            jax.ShapeDtypeStruct((N, D), jnp.float32),
            jax.ShapeDtypeStruct((N, 1), jnp.float32),
        ],
    )(x, W1, d0, d1)


def _combine_body(final, s_parts_ref0, s_parts_ref1, h_ref, inv_ref, b_ref,
                  w_ref, bo_ref, o_ref):
    s_tot = s_parts_ref0[0] + s_parts_ref1[0] + h_ref[...]
    z = jnp.maximum(s_tot * inv_ref[...] + b_ref[...], 0.0)
    o = jnp.dot(z, w_ref[...], preferred_element_type=jnp.float32)
    if final:
        o_ref[...] = o + bo_ref[...]
    else:
        o_ref[...] = o * inv_ref[...]


def _combine(s_parts, h, inv, b, w, bo, final):
    d_out = w.shape[1]
    return pl.pallas_call(
        functools.partial(_combine_body, final),
        grid=(_GRID,),
        in_specs=[
            pl.BlockSpec((1, _BR, D), lambda i: (0, i, 0)),
            pl.BlockSpec((1, _BR, D), lambda i: (1, i, 0)),
            pl.BlockSpec((_BR, D), lambda i: (i, 0)),
            pl.BlockSpec((_BR, 1), lambda i: (i, 0)),
            pl.BlockSpec((1, D), lambda i: (0, 0)),
            pl.BlockSpec((D, d_out), lambda i: (0, 0)),
            pl.BlockSpec((1, d_out), lambda i: (0, 0)),
        ],
        out_specs=pl.BlockSpec((_BR, d_out), lambda i: (i, 0)),
        out_shape=jax.ShapeDtypeStruct((N, d_out), jnp.float32),
    )(s_parts, s_parts, h, inv, b, w, bo)


def kernel(x, edge_index, W1, b1, W2, b2, W3, b3, Wo, bo):
    src = edge_index[0].astype(jnp.int32)
    dst = edge_index[1].astype(jnp.int32)

    deg_parts = _deg_kernel(dst)
    d0 = deg_parts[:N][:, None]
    d1 = deg_parts[N:][:, None]

    h1, inv = _first_layer(x, W1, d0, d1)

    b1r = b1[None, :]
    b2r = b2[None, :]
    b3r = b3[None, :]
    bor = bo[None, :]
    dummy = jnp.zeros((1, D), jnp.float32)

    s1 = _prop_kernel(h1, src, dst)
    h2 = _combine(s1, h1, inv, b1r, W2, dummy, final=False)
    s2 = _prop_kernel(h2, src, dst)
    h3 = _combine(s2, h2, inv, b2r, W3, dummy, final=False)
    s3 = _prop_kernel(h3, src, dst)
    out = _combine(s3, h3, inv, b3r, Wo, bor, final=True)
    return out


# speedup vs baseline: 28.0552x; 1.3832x over previous
import functools

import jax
import jax.numpy as jnp
from jax import lax
from jax.experimental import pallas as pl
from jax.experimental.pallas import tpu as pltpu
from jax.experimental.pallas import tpu_sc as plsc

N = 10000
E = 320000
D = 128
D_OUT = 64

NC = 2
NS = 16
CH = 80
EPW = E // (NC * NS)
NCHUNK = EPW // CH

RPT = 624
NZCH = 8


def _chunk_starts(s):
    return [jnp.minimum(s * RPT + t * CH, N - CH) for t in range(NZCH)]


_mesh = plsc.VectorSubcoreMesh(core_axis_name="c", subcore_axis_name="s")


@functools.partial(
    pl.kernel,
    out_type=jax.ShapeDtypeStruct((NC * N,), jnp.float32),
    mesh=_mesh,
    scratch_types=[
        pltpu.VMEM((CH,), jnp.int32),
        pltpu.VMEM((CH,), jnp.int32),
        pltpu.VMEM((CH,), jnp.float32),
        pltpu.VMEM((CH,), jnp.float32),
        pltpu.VMEM_SHARED((N,), jnp.float32),
        pltpu.SemaphoreType.DMA,
    ],
)
def _deg_kernel(dst_hbm, out_hbm, dst_a, dst_b, ones_v, zero_v, acc, sem):
    c = lax.axis_index("c")
    s = lax.axis_index("s")

    for j in range(CH // 16):
        ones_v[pl.ds(j * 16, 16)] = jnp.ones((16,), jnp.float32)
        zero_v[pl.ds(j * 16, 16)] = jnp.zeros((16,), jnp.float32)

    for st in _chunk_starts(s):
        pltpu.sync_copy(zero_v, acc.at[pl.ds(st, CH)])

    plsc.subcore_barrier()

    base = (c * NS + s) * EPW

    def idx_start(k, dv):
        pltpu.async_copy(dst_hbm.at[pl.ds(base + k * CH, CH)], dv, sem)

    def idx_wait(dv):
        pltpu.make_async_copy(dst_hbm.at[pl.ds(0, CH)], dv, sem).wait()

    idx_start(0, dst_a)
    idx_start(1, dst_b)

    def pair(g, carry):
        k0 = 2 * g
        idx_wait(dst_a)
        pltpu.sync_copy(ones_v, acc.at[dst_a], add=True)
        idx_start(k0 + 2, dst_a)
        idx_wait(dst_b)
        pltpu.sync_copy(ones_v, acc.at[dst_b], add=True)

        @pl.when(g < (NCHUNK - 1) // 2 - 1)
        def _():
            idx_start(k0 + 3, dst_b)

        return carry

    lax.fori_loop(0, (NCHUNK - 1) // 2, pair, 0)

    idx_wait(dst_a)
    pltpu.sync_copy(ones_v, acc.at[dst_a], add=True)
    plsc.subcore_barrier()

    for st in _chunk_starts(s):
        pltpu.sync_copy(acc.at[pl.ds(st, CH)], zero_v)
        pltpu.sync_copy(zero_v, out_hbm.at[pl.ds(c * N + st, CH)])


_NSLOT = 3


@functools.partial(
    pl.kernel,
    out_type=jax.ShapeDtypeStruct((NC, N, D), jnp.float32),
    mesh=_mesh,
    scratch_types=[
        pltpu.VMEM((_NSLOT, CH), jnp.int32),
        pltpu.VMEM((_NSLOT + 1, CH), jnp.int32),
        pltpu.VMEM((_NSLOT, CH, D), jnp.float32),
        pltpu.VMEM_SHARED((N, D), jnp.float32),
        pltpu.SemaphoreType.DMA,
        pltpu.SemaphoreType.DMA,
        pltpu.SemaphoreType.DMA,
    ],
)
def _prop_kernel(h_hbm, src_hbm, dst_hbm, out_hbm,
                 src_i, dst_i, rows_all, acc, sem_g, sem_s, sem_i):
    c = lax.axis_index("c")
    s = lax.axis_index("s")
    rows = [rows_all.at[r] for r in range(_NSLOT)]

    def zrow(k, carry):
        for j in range(D // 16):
            rows_all[0, k, pl.ds(j * 16, 16)] = jnp.zeros((16,), jnp.float32)
        return carry

    lax.fori_loop(0, CH, zrow, 0)

    for st in _chunk_starts(s):
        pltpu.sync_copy(rows[0], acc.at[pl.ds(st, CH)])

    plsc.subcore_barrier()

    base = (c * NS + s) * EPW

    def idx_start(k, rs, rd):
        off = base + k * CH
        pltpu.async_copy(src_hbm.at[pl.ds(off, CH)], src_i.at[rs], sem_i)
        pltpu.async_copy(dst_hbm.at[pl.ds(off, CH)], dst_i.at[rd], sem_i)

    def idx_wait():
        pltpu.make_async_copy(src_hbm.at[pl.ds(0, CH)], src_i.at[0], sem_i).wait()
        pltpu.make_async_copy(src_hbm.at[pl.ds(0, CH)], src_i.at[0], sem_i).wait()

    def gather_start(rs, r):
        pltpu.async_copy(h_hbm.at[src_i.at[rs]], rows[r], sem_g)

    def gather_wait(r):
        pltpu.make_async_copy(h_hbm.at[src_i.at[0]], rows[r], sem_g).wait()

    def scatter_start(rd, r):
        pltpu.async_copy(rows[r], acc.at[dst_i.at[rd]], sem_s, add=True)

    def scatter_wait(r):
        pltpu.make_async_copy(rows[r], acc.at[dst_i.at[0]], sem_s).wait()

    def chunk(k, j, first, do_gnext, do_inext):
        r = j % _NSLOT
        if not first:
            scatter_wait((r + 2) % _NSLOT)
        if do_gnext:
            idx_wait()
            gather_start((j + 2) % _NSLOT, (r + 2) % _NSLOT)
        gather_wait(r)
        if do_inext:
            idx_start(k + 3, (j + 3) % _NSLOT, (j + 3) % (_NSLOT + 1))
        scatter_start(j % (_NSLOT + 1), r)

    pltpu.sync_copy(src_hbm.at[pl.ds(base, CH)], src_i.at[0])
    pltpu.sync_copy(dst_hbm.at[pl.ds(base, CH)], dst_i.at[0])
    pltpu.sync_copy(src_hbm.at[pl.ds(base + CH, CH)], src_i.at[1])
    pltpu.sync_copy(dst_hbm.at[pl.ds(base + CH, CH)], dst_i.at[1])
    gather_start(0, 0)
    gather_start(1, 1)
    idx_start(2, 2, 2)

    for k in range(12):
        chunk(k, k, k == 0, True, True)

    def body(g, carry):
        k0 = 12 * g
        for j in range(12):
            chunk(k0 + j, j, False, True, True)
        return carry

    lax.fori_loop(1, (NCHUNK - 5) // 12, body, 0)

    chunk(120, 0, False, True, True)
    chunk(121, 1, False, True, True)
    chunk(122, 2, False, True, False)
    chunk(123, 3, False, False, False)
    chunk(124, 4, False, False, False)
    scatter_wait(1)

    plsc.subcore_barrier()

    for st in _chunk_starts(s):
        pltpu.sync_copy(acc.at[pl.ds(st, CH)], rows[0])
        pltpu.sync_copy(rows[0], out_hbm.at[c, pl.ds(st, CH)])


_BR = 1000
_GRID = N // _BR


def _mm_scale_body(x_ref, w_ref, d0_ref, d1_ref, h_ref, inv_ref):
    inv = lax.rsqrt(d0_ref[...] + d1_ref[...] + 1.0)
    h = jnp.dot(x_ref[...], w_ref[...], preferred_element_type=jnp.float32)
    h_ref[...] = h * inv
    inv_ref[...] = inv


def _first_layer(x, W1, d0, d1):
    return pl.pallas_call(
        _mm_scale_body,
        grid=(_GRID,),
        in_specs=[
            pl.BlockSpec((_BR, D), lambda i: (i, 0)),
            pl.BlockSpec((D, D), lambda i: (0, 0)),
            pl.BlockSpec((_BR, 1), lambda i: (i, 0)),
            pl.BlockSpec((_BR, 1), lambda i: (i, 0)),
        ],
        out_specs=[
            pl.BlockSpec((_BR, D), lambda i: (i, 0)),
            pl.BlockSpec((_BR, 1), lambda i: (i, 0)),
        ],
        out_shape=[
            jax.ShapeDtypeStruct((N, D), jnp.float32),
            jax.ShapeDtypeStruct((N, 1), jnp.float32),
        ],
    )(x, W1, d0, d1)


def _combine_body(final, s_parts_ref0, s_parts_ref1, h_ref, inv_ref, b_ref,
                  w_ref, bo_ref, o_ref):
    s_tot = s_parts_ref0[0] + s_parts_ref1[0] + h_ref[...]
    z = jnp.maximum(s_tot * inv_ref[...] + b_ref[...], 0.0)
    o = jnp.dot(z, w_ref[...], preferred_element_type=jnp.float32)
    if final:
        o_ref[...] = o + bo_ref[...]
    else:
        o_ref[...] = o * inv_ref[...]


def _combine(s_parts, h, inv, b, w, bo, final):
    d_out = w.shape[1]
    return pl.pallas_call(
        functools.partial(_combine_body, final),
        grid=(_GRID,),
        in_specs=[
            pl.BlockSpec((1, _BR, D), lambda i: (0, i, 0)),
            pl.BlockSpec((1, _BR, D), lambda i: (1, i, 0)),
            pl.BlockSpec((_BR, D), lambda i: (i, 0)),
            pl.BlockSpec((_BR, 1), lambda i: (i, 0)),
            pl.BlockSpec((1, D), lambda i: (0, 0)),
            pl.BlockSpec((D, d_out), lambda i: (0, 0)),
            pl.BlockSpec((1, d_out), lambda i: (0, 0)),
        ],
        out_specs=pl.BlockSpec((_BR, d_out), lambda i: (i, 0)),
        out_shape=jax.ShapeDtypeStruct((N, d_out), jnp.float32),
    )(s_parts, s_parts, h, inv, b, w, bo)


def kernel(x, edge_index, W1, b1, W2, b2, W3, b3, Wo, bo):
    src = edge_index[0].astype(jnp.int32)
    dst = edge_index[1].astype(jnp.int32)

    deg_parts = _deg_kernel(dst)
    d0 = deg_parts[:N][:, None]
    d1 = deg_parts[N:][:, None]

    h1, inv = _first_layer(x, W1, d0, d1)

    b1r = b1[None, :]
    b2r = b2[None, :]
    b3r = b3[None, :]
    bor = bo[None, :]
    dummy = jnp.zeros((1, D), jnp.float32)

    s1 = _prop_kernel(h1, src, dst)
    h2 = _combine(s1, h1, inv, b1r, W2, dummy, final=False)
    s2 = _prop_kernel(h2, src, dst)
    h3 = _combine(s2, h2, inv, b2r, W3, dummy, final=False)
    s3 = _prop_kernel(h3, src, dst)
    out = _combine(s3, h3, inv, b3r, Wo, bor, final=True)
    return out


# speedup vs baseline: 29.4437x; 1.0495x over previous
import functools

import jax
import jax.numpy as jnp
from jax import lax
from jax.experimental import pallas as pl
from jax.experimental.pallas import tpu as pltpu
from jax.experimental.pallas import tpu_sc as plsc

N = 10000
E = 320000
D = 128
D_OUT = 64

NC = 2
NS = 16
CH = 80
EPW = E // (NC * NS)
NCHUNK = EPW // CH

RPT = 624
NZCH = 8


def _chunk_starts(s):
    return [jnp.minimum(s * RPT + t * CH, N - CH) for t in range(NZCH)]


_mesh = plsc.VectorSubcoreMesh(core_axis_name="c", subcore_axis_name="s")


@functools.partial(
    pl.kernel,
    out_type=jax.ShapeDtypeStruct((NC * N,), jnp.float32),
    mesh=_mesh,
    scratch_types=[
        pltpu.VMEM((8, CH), jnp.int32),
        pltpu.VMEM((CH,), jnp.float32),
        pltpu.VMEM((CH,), jnp.float32),
        pltpu.VMEM_SHARED((N,), jnp.float32),
        pltpu.SemaphoreType.DMA,
        pltpu.SemaphoreType.DMA,
    ],
)
def _deg_kernel(dst_hbm, out_hbm, dst_i, ones_v, zero_v, acc, sem_s, sem_i):
    c = lax.axis_index("c")
    s = lax.axis_index("s")

    for j in range(CH // 16):
        ones_v[pl.ds(j * 16, 16)] = jnp.ones((16,), jnp.float32)
        zero_v[pl.ds(j * 16, 16)] = jnp.zeros((16,), jnp.float32)

    starts = _chunk_starts(s)
    for st in starts:
        pltpu.async_copy(zero_v, acc.at[pl.ds(st, CH)], sem_s)
    for st in starts:
        pltpu.make_async_copy(zero_v, acc.at[pl.ds(st, CH)], sem_s).wait()

    plsc.subcore_barrier()

    base = (c * NS + s) * EPW

    def idx_start(k, rd):
        pltpu.async_copy(dst_hbm.at[pl.ds(base + k * CH, CH)], dst_i.at[rd],
                         sem_i)

    def idx_wait():
        pltpu.make_async_copy(dst_hbm.at[pl.ds(0, CH)], dst_i.at[0],
                              sem_i).wait()

    def sc_start(rd):
        pltpu.async_copy(ones_v, acc.at[dst_i.at[rd]], sem_s, add=True)

    def sc_wait():
        pltpu.make_async_copy(ones_v, acc.at[dst_i.at[0]], sem_s).wait()

    def chunk(k, j, do_wait, do_inext):
        if do_wait:
            sc_wait()
        idx_wait()
        sc_start(j % 8)
        if do_inext:
            idx_start(k + 3, (j + 3) % 8)

    idx_start(0, 0)
    idx_start(1, 1)
    idx_start(2, 2)

    for k in range(8):
        chunk(k, k, k >= 4, True)

    def body(g, carry):
        k0 = 8 * g
        for j in range(8):
            chunk(k0 + j, j, True, True)
        return carry

    lax.fori_loop(1, 15, body, 0)

    chunk(120, 0, True, True)
    chunk(121, 1, True, True)
    chunk(122, 2, True, False)
    chunk(123, 3, True, False)
    chunk(124, 4, True, False)
    for _ in range(4):
        sc_wait()

    plsc.subcore_barrier()

    stage = [zero_v, ones_v]
    for t, st in enumerate(starts):
        if t >= 2:
            pltpu.make_async_copy(stage[t % 2],
                                  out_hbm.at[pl.ds(0, CH)], sem_i).wait()
        pltpu.sync_copy(acc.at[pl.ds(st, CH)], stage[t % 2])
        pltpu.async_copy(stage[t % 2], out_hbm.at[pl.ds(c * N + st, CH)], sem_i)
    for t in range(2):
        pltpu.make_async_copy(stage[t], out_hbm.at[pl.ds(0, CH)], sem_i).wait()


_NSLOT = 3


@functools.partial(
    pl.kernel,
    out_type=jax.ShapeDtypeStruct((NC, N, D), jnp.float32),
    mesh=_mesh,
    scratch_types=[
        pltpu.VMEM((_NSLOT, CH), jnp.int32),
        pltpu.VMEM((_NSLOT + 1, CH), jnp.int32),
        pltpu.VMEM((_NSLOT, CH, D), jnp.float32),
        pltpu.VMEM_SHARED((N, D), jnp.float32),
        pltpu.SemaphoreType.DMA,
        pltpu.SemaphoreType.DMA,
        pltpu.SemaphoreType.DMA,
    ],
)
def _prop_kernel(h_hbm, src_hbm, dst_hbm, out_hbm,
                 src_i, dst_i, rows_all, acc, sem_g, sem_s, sem_i):
    c = lax.axis_index("c")
    s = lax.axis_index("s")
    rows = [rows_all.at[r] for r in range(_NSLOT)]

    def zrow(k, carry):
        for j in range(D // 16):
            rows_all[0, k, pl.ds(j * 16, 16)] = jnp.zeros((16,), jnp.float32)
        return carry

    lax.fori_loop(0, CH, zrow, 0)

    for st in _chunk_starts(s):
        pltpu.async_copy(rows[0], acc.at[pl.ds(st, CH)], sem_s)
    for st in _chunk_starts(s):
        pltpu.make_async_copy(rows[0], acc.at[pl.ds(st, CH)], sem_s).wait()

    plsc.subcore_barrier()

    base = (c * NS + s) * EPW

    def idx_start(k, rs, rd):
        off = base + k * CH
        pltpu.async_copy(src_hbm.at[pl.ds(off, CH)], src_i.at[rs], sem_i)
        pltpu.async_copy(dst_hbm.at[pl.ds(off, CH)], dst_i.at[rd], sem_i)

    def idx_wait():
        pltpu.make_async_copy(src_hbm.at[pl.ds(0, CH)], src_i.at[0], sem_i).wait()
        pltpu.make_async_copy(src_hbm.at[pl.ds(0, CH)], src_i.at[0], sem_i).wait()

    def gather_start(rs, r):
        pltpu.async_copy(h_hbm.at[src_i.at[rs]], rows[r], sem_g)

    def gather_wait(r):
        pltpu.make_async_copy(h_hbm.at[src_i.at[0]], rows[r], sem_g).wait()

    def scatter_start(rd, r):
        pltpu.async_copy(rows[r], acc.at[dst_i.at[rd]], sem_s, add=True)

    def scatter_wait(r):
        pltpu.make_async_copy(rows[r], acc.at[dst_i.at[0]], sem_s).wait()

    def chunk(k, j, first, do_gnext, do_inext):
        r = j % _NSLOT
        if not first:
            scatter_wait((r + 2) % _NSLOT)
        if do_gnext:
            idx_wait()
            gather_start((j + 2) % _NSLOT, (r + 2) % _NSLOT)
        gather_wait(r)
        if do_inext:
            idx_start(k + 3, (j + 3) % _NSLOT, (j + 3) % (_NSLOT + 1))
        scatter_start(j % (_NSLOT + 1), r)

    pltpu.sync_copy(src_hbm.at[pl.ds(base, CH)], src_i.at[0])
    pltpu.sync_copy(dst_hbm.at[pl.ds(base, CH)], dst_i.at[0])
    pltpu.sync_copy(src_hbm.at[pl.ds(base + CH, CH)], src_i.at[1])
    pltpu.sync_copy(dst_hbm.at[pl.ds(base + CH, CH)], dst_i.at[1])
    gather_start(0, 0)
    gather_start(1, 1)
    idx_start(2, 2, 2)

    for k in range(12):
        chunk(k, k, k == 0, True, True)

    def body(g, carry):
        k0 = 12 * g
        for j in range(12):
            chunk(k0 + j, j, False, True, True)
        return carry

    lax.fori_loop(1, (NCHUNK - 5) // 12, body, 0)

    chunk(120, 0, False, True, True)
    chunk(121, 1, False, True, True)
    chunk(122, 2, False, True, False)
    chunk(123, 3, False, False, False)
    chunk(124, 4, False, False, False)
    scatter_wait(1)

    plsc.subcore_barrier()

    for t, st in enumerate(_chunk_starts(s)):
        if t >= 2:
            pltpu.make_async_copy(rows[t % 2],
                                  out_hbm.at[c, pl.ds(0, CH)], sem_g).wait()
        pltpu.sync_copy(acc.at[pl.ds(st, CH)], rows[t % 2])
        pltpu.async_copy(rows[t % 2], out_hbm.at[c, pl.ds(st, CH)], sem_g)
    for t in range(2):
        pltpu.make_async_copy(rows[t], out_hbm.at[c, pl.ds(0, CH)], sem_g).wait()


_BR = 1000
_GRID = N // _BR


def _mm_scale_body(x_ref, w_ref, d0_ref, d1_ref, h_ref, inv_ref):
    inv = lax.rsqrt(d0_ref[...] + d1_ref[...] + 1.0)
    h = jnp.dot(x_ref[...], w_ref[...], preferred_element_type=jnp.float32)
    h_ref[...] = h * inv
    inv_ref[...] = inv


def _first_layer(x, W1, d0, d1):
    return pl.pallas_call(
        _mm_scale_body,
        grid=(_GRID,),
        in_specs=[
            pl.BlockSpec((_BR, D), lambda i: (i, 0)),
            pl.BlockSpec((D, D), lambda i: (0, 0)),
            pl.BlockSpec((_BR, 1), lambda i: (i, 0)),
            pl.BlockSpec((_BR, 1), lambda i: (i, 0)),
        ],
        out_specs=[
            pl.BlockSpec((_BR, D), lambda i: (i, 0)),
            pl.BlockSpec((_BR, 1), lambda i: (i, 0)),
        ],
        out_shape=[
            jax.ShapeDtypeStruct((N, D), jnp.float32),
            jax.ShapeDtypeStruct((N, 1), jnp.float32),
        ],
    )(x, W1, d0, d1)


def _combine_body(final, s_parts_ref0, s_parts_ref1, h_ref, inv_ref, b_ref,
                  w_ref, bo_ref, o_ref):
    s_tot = s_parts_ref0[0] + s_parts_ref1[0] + h_ref[...]
    z = jnp.maximum(s_tot * inv_ref[...] + b_ref[...], 0.0)
    o = jnp.dot(z, w_ref[...], preferred_element_type=jnp.float32)
    if final:
        o_ref[...] = o + bo_ref[...]
    else:
        o_ref[...] = o * inv_ref[...]


def _combine(s_parts, h, inv, b, w, bo, final):
    d_out = w.shape[1]
    return pl.pallas_call(
        functools.partial(_combine_body, final),
        grid=(_GRID,),
        in_specs=[
            pl.BlockSpec((1, _BR, D), lambda i: (0, i, 0)),
            pl.BlockSpec((1, _BR, D), lambda i: (1, i, 0)),
            pl.BlockSpec((_BR, D), lambda i: (i, 0)),
            pl.BlockSpec((_BR, 1), lambda i: (i, 0)),
            pl.BlockSpec((1, D), lambda i: (0, 0)),
            pl.BlockSpec((D, d_out), lambda i: (0, 0)),
            pl.BlockSpec((1, d_out), lambda i: (0, 0)),
        ],
        out_specs=pl.BlockSpec((_BR, d_out), lambda i: (i, 0)),
        out_shape=jax.ShapeDtypeStruct((N, d_out), jnp.float32),
    )(s_parts, s_parts, h, inv, b, w, bo)


def kernel(x, edge_index, W1, b1, W2, b2, W3, b3, Wo, bo):
    src = edge_index[0].astype(jnp.int32)
    dst = edge_index[1].astype(jnp.int32)

    deg_parts = _deg_kernel(dst)
    d0 = deg_parts[:N][:, None]
    d1 = deg_parts[N:][:, None]

    h1, inv = _first_layer(x, W1, d0, d1)

    b1r = b1[None, :]
    b2r = b2[None, :]
    b3r = b3[None, :]
    bor = bo[None, :]
    dummy = jnp.zeros((1, D), jnp.float32)

    s1 = _prop_kernel(h1, src, dst)
    h2 = _combine(s1, h1, inv, b1r, W2, dummy, final=False)
    s2 = _prop_kernel(h2, src, dst)
    h3 = _combine(s2, h2, inv, b2r, W3, dummy, final=False)
    s3 = _prop_kernel(h3, src, dst)
    out = _combine(s3, h3, inv, b3r, Wo, bor, final=True)
    return out
